# baseline (speedup 1.0000x reference)
import functools

import jax
import jax.numpy as jnp
from jax import lax
from jax.experimental import pallas as pl
from jax.experimental.pallas import tpu as pltpu
from jax.experimental.pallas import tpu_sc as plsc

BATCH = 4096
SEQ = 50
DIM = 32
NC = 2
NS = 16
NW = NC * NS
TOTAL = BATCH * SEQ
B_PER_W = TOTAL // NW
CHUNK = 128
NCHUNK = B_PER_W // CHUNK

_mesh = plsc.VectorSubcoreMesh(core_axis_name="c", subcore_axis_name="s")


@functools.partial(
    pl.kernel,
    mesh=_mesh,
    out_type=jax.ShapeDtypeStruct((TOTAL, DIM), jnp.float32),
    scratch_types=[
        pltpu.VMEM((NCHUNK, CHUNK), jnp.int32),
        pltpu.VMEM((CHUNK, DIM), jnp.float32),
        pltpu.SemaphoreType.DMA,
    ],
    compiler_params=pltpu.CompilerParams(use_tc_tiling_on_sc=False),
)
def _gather_kernel(idx_hbm, tab_hbm, out_hbm, idx_v, rows_v, sem):
    wid = lax.axis_index("s") * NC + lax.axis_index("c")
    base = wid * B_PER_W
    pltpu.sync_copy(idx_hbm.at[wid], idx_v)

    def body(i, carry):
        pltpu.async_copy(tab_hbm.at[idx_v.at[i]], rows_v, sem).wait()
        pltpu.sync_copy(rows_v, out_hbm.at[pl.ds(base + i * CHUNK, CHUNK)])
        return carry

    lax.fori_loop(0, NCHUNK, body, 0)


def kernel(x, weight):
    idx = x.reshape(NW, NCHUNK, CHUNK).astype(jnp.int32)
    out = _gather_kernel(idx, weight)
    return out.reshape(BATCH, SEQ, DIM)


# speedup vs baseline: 1.2836x; 1.2836x over previous
import functools

import jax
import jax.numpy as jnp
from jax import lax
from jax.experimental import pallas as pl
from jax.experimental.pallas import tpu as pltpu
from jax.experimental.pallas import tpu_sc as plsc

BATCH = 4096
SEQ = 50
DIM = 32
NC = 2
NS = 16
NW = NC * NS
BBLK = BATCH // NW
NBUF = 4

_mesh = plsc.VectorSubcoreMesh(core_axis_name="c", subcore_axis_name="s")


@functools.partial(
    pl.kernel,
    mesh=_mesh,
    out_type=jax.ShapeDtypeStruct((BATCH, SEQ, DIM), jnp.float32),
    scratch_types=[
        pltpu.VMEM((SEQ, BBLK), jnp.int32),
        pltpu.VMEM((NBUF, BBLK, DIM), jnp.float32),
        pltpu.SemaphoreType.DMA,
    ],
    compiler_params=pltpu.CompilerParams(use_tc_tiling_on_sc=False),
)
def _gather_kernel(xt_hbm, tab_hbm, out_hbm, idx_v, rows_v, gsem):
    wid = lax.axis_index("s") * NC + lax.axis_index("c")
    b0 = wid * BBLK
    pltpu.sync_copy(xt_hbm.at[:, pl.ds(b0, BBLK)], idx_v)

    for k in range(NBUF - 1):
        pltpu.async_copy(tab_hbm.at[idx_v.at[k]], rows_v.at[k], gsem)

    def body(s, carry):
        pltpu.async_copy(
            tab_hbm.at[idx_v.at[s]], rows_v.at[lax.rem(s, NBUF)], gsem
        )
        pltpu.make_async_copy(
            tab_hbm.at[idx_v.at[0]], rows_v.at[0], gsem
        ).wait()
        old = lax.rem(s + 1, NBUF)
        pltpu.sync_copy(
            rows_v.at[old], out_hbm.at[pl.ds(b0, BBLK), s - (NBUF - 1)]
        )
        return carry

    lax.fori_loop(NBUF - 1, SEQ, body, 0)

    for k in range(SEQ - NBUF + 1, SEQ):
        pltpu.make_async_copy(
            tab_hbm.at[idx_v.at[0]], rows_v.at[0], gsem
        ).wait()
        pltpu.sync_copy(
            rows_v.at[k % NBUF], out_hbm.at[pl.ds(b0, BBLK), k]
        )


def kernel(x, weight):
    return _gather_kernel(x.T, weight)
